# baseline (speedup 1.0000x reference)
import functools

import jax
import jax.numpy as jnp
from jax import lax
from jax.experimental import pallas as pl
from jax.experimental.pallas import tpu as pltpu
from jax.experimental.pallas import tpu_sc as plsc

N = 50000
DV = 256
DG = 256
S = 128

NC = 2
NS = 16
NW = NC * NS
SEG_PER_W = S // NW
LANES = 16
NJ = DV // LANES
GROUPS = 2
JPG = NJ // GROUPS
CH = 64
NBUF = 2
OFF_PAD = 144


def _sc_segment_stats(fv, offsets):
    mesh = plsc.VectorSubcoreMesh(core_axis_name="c", subcore_axis_name="s")
    out_type = tuple(
        jax.ShapeDtypeStruct((S * DV,), jnp.float32) for _ in range(3))

    @functools.partial(
        pl.kernel,
        mesh=mesh,
        out_type=out_type,
        scratch_types=[
            pltpu.VMEM((OFF_PAD,), jnp.int32),
            pltpu.VMEM((CH, DV), jnp.float32),
            pltpu.VMEM((CH, DV), jnp.float32),
            pltpu.VMEM((SEG_PER_W * DV,), jnp.float32),
            pltpu.VMEM((SEG_PER_W * DV,), jnp.float32),
            pltpu.VMEM((SEG_PER_W * DV,), jnp.float32),
            pltpu.SemaphoreType.DMA,
            pltpu.SemaphoreType.DMA,
        ],
    )
    def k(fv_hbm, off_hbm, mean_hbm, mn_hbm, mx_hbm,
          off_v, buf0, buf1, s_v, mn_v, mx_v,
          sem0, sem1):
        bufs = (buf0, buf1)
        sems = (sem0, sem1)
        wid = lax.axis_index("s") * NC + lax.axis_index("c")
        pltpu.sync_copy(off_hbm, off_v)

        for kk in range(SEG_PER_W):
            seg = wid * SEG_PER_W + kk
            offv = off_v[pl.ds(seg, LANES)]
            a = offv[0]
            b = offv[1]
            n = b - a
            a8 = (a // 8) * 8
            nch = (b - a8 + CH - 1) // CH
            nquad = (nch + NBUF - 1) // NBUF

            def issue(c, buf, sem, a8=a8):
                row0 = jnp.minimum(a8 + c * CH, N - CH)
                pltpu.async_copy(fv_hbm.at[pl.ds(row0, CH)], buf, sem)

            def wait(buf, sem):
                pltpu.make_async_copy(
                    fv_hbm.at[pl.ds(0, CH)], buf, sem).wait()

            def process(buf, c, cy, a=a, b=b, a8=a8):
                row0 = a8 + c * CH
                w0 = jnp.minimum(row0, N - CH)
                lo = jnp.maximum(a, row0) - w0
                hi = jnp.maximum(jnp.minimum(b, row0 + CH) - w0, lo)
                cy = list(cy)
                for g in range(GROUPS):
                    sub = tuple(cy[3 * JPG * g: 3 * JPG * (g + 1)])

                    def row_body(r, sc, g=g, buf=buf):
                        out = []
                        for jj in range(JPG):
                            j = JPG * g + jj
                            v = buf[r, pl.ds(LANES * j, LANES)]
                            out += [
                                sc[3 * jj] + v,
                                jnp.minimum(sc[3 * jj + 1], v),
                                jnp.maximum(sc[3 * jj + 2], v),
                            ]
                        return tuple(out)

                    sub = plsc.parallel_loop(
                        lo, hi, 1, unroll=8, carry=sub)(row_body)
                    cy[3 * JPG * g: 3 * JPG * (g + 1)] = list(sub)
                return tuple(cy)

            carry = []
            for _ in range(NJ):
                carry += [
                    jnp.zeros((LANES,), jnp.float32),
                    jnp.full((LANES,), jnp.inf, jnp.float32),
                    jnp.full((LANES,), -jnp.inf, jnp.float32),
                ]
            carry = tuple(carry)

            for i in range(NBUF - 1):
                @pl.when(i < nch)
                def _(i=i):
                    issue(i, bufs[i], sems[i])

            def quad_body(q, cy):
                for i in range(NBUF):
                    c = NBUF * q + i

                    @pl.when(c < nch)
                    def _(c=c, i=i):
                        wait(bufs[i], sems[i])

                    @pl.when(c + (NBUF - 1) < nch)
                    def _(c=c, i=i):
                        issue(c + NBUF - 1, bufs[(i + NBUF - 1) % NBUF],
                              sems[(i + NBUF - 1) % NBUF])

                    cy = process(bufs[i], c, cy)
                return cy

            carry = lax.fori_loop(0, nquad, quad_body, carry)

            nf = n.astype(jnp.float32)
            inv = 1.0 / jnp.maximum(jnp.broadcast_to(nf, (LANES,)), 1.0)

            @pl.when(n > 0)
            def _(carry=carry, inv=inv, kk=kk):
                for j in range(NJ):
                    ds = pl.ds(kk * DV + LANES * j, LANES)
                    s_v[ds] = carry[3 * j] * inv
                    mn_v[ds] = carry[3 * j + 1]
                    mx_v[ds] = carry[3 * j + 2]

            @pl.when(n == 0)
            def _(kk=kk):
                z = jnp.zeros((LANES,), jnp.float32)
                for j in range(NJ):
                    ds = pl.ds(kk * DV + LANES * j, LANES)
                    s_v[ds] = z
                    mn_v[ds] = z
                    mx_v[ds] = z

        wbase = wid * SEG_PER_W * DV
        wlen = SEG_PER_W * DV
        pltpu.sync_copy(s_v, mean_hbm.at[pl.ds(wbase, wlen)])
        pltpu.sync_copy(mn_v, mn_hbm.at[pl.ds(wbase, wlen)])
        pltpu.sync_copy(mx_v, mx_hbm.at[pl.ds(wbase, wlen)])

    return k(fv, offsets)


def _tc_combine(mean, mn, mx, W1, W2, W3, bsum):
    def body(mean_ref, mn_ref, mx_ref, w1_ref, w2_ref, w3_ref, b_ref, o_ref):
        dn = (((1,), (1,)), ((), ()))
        acc = lax.dot_general(mean_ref[...], w1_ref[...], dn,
                              precision=lax.Precision.HIGHEST,
                              preferred_element_type=jnp.float32)
        acc = acc + lax.dot_general(mn_ref[...], w2_ref[...], dn,
                                    precision=lax.Precision.HIGHEST,
                                    preferred_element_type=jnp.float32)
        acc = acc + lax.dot_general(mx_ref[...], w3_ref[...], dn,
                                    precision=lax.Precision.HIGHEST,
                                    preferred_element_type=jnp.float32)
        o_ref[...] = acc + b_ref[...]

    return pl.pallas_call(
        body,
        out_shape=jax.ShapeDtypeStruct((S, DG), jnp.float32),
    )(mean, mn, mx, W1, W2, W3, bsum)


def kernel(fv, segment_ids, W1, b1, W2, b2, W3, b3):
    ids = segment_ids.astype(jnp.int32)
    off = jnp.searchsorted(ids, jnp.arange(S + 1, dtype=jnp.int32),
                           side="left", method="compare_all").astype(jnp.int32)
    off = jnp.concatenate([off, jnp.zeros((OFF_PAD - (S + 1),), jnp.int32)])
    mean, mn, mx = _sc_segment_stats(fv, off)
    bsum = (b1 + b2 + b3).reshape(1, DG)
    return _tc_combine(mean.reshape(S, DV), mn.reshape(S, DV),
                       mx.reshape(S, DV), W1, W2, W3, bsum)


# speedup vs baseline: 1.0721x; 1.0721x over previous
import functools

import jax
import jax.numpy as jnp
from jax import lax
from jax.experimental import pallas as pl
from jax.experimental.pallas import tpu as pltpu
from jax.experimental.pallas import tpu_sc as plsc

N = 50000
DV = 256
DG = 256
S = 128

NC = 2
NS = 16
NW = NC * NS
SEG_PER_W = S // NW
LANES = 16
NJ = DV // LANES
GROUPS = 2
JPG = NJ // GROUPS
CH = 64
NBUF = 2
OFF_PAD = 144


def _sc_segment_stats(fv, offsets):
    mesh = plsc.VectorSubcoreMesh(core_axis_name="c", subcore_axis_name="s")
    out_type = tuple(
        jax.ShapeDtypeStruct((S * DV,), jnp.float32) for _ in range(3))

    @functools.partial(
        pl.kernel,
        mesh=mesh,
        out_type=out_type,
        scratch_types=[
            pltpu.VMEM((OFF_PAD,), jnp.int32),
            pltpu.VMEM((CH, DV), jnp.float32),
            pltpu.VMEM((CH, DV), jnp.float32),
            pltpu.VMEM((SEG_PER_W * DV,), jnp.float32),
            pltpu.VMEM((SEG_PER_W * DV,), jnp.float32),
            pltpu.VMEM((SEG_PER_W * DV,), jnp.float32),
            pltpu.SemaphoreType.DMA,
            pltpu.SemaphoreType.DMA,
        ],
    )
    def k(fv_hbm, off_hbm, mean_hbm, mn_hbm, mx_hbm,
          off_v, buf0, buf1, s_v, mn_v, mx_v,
          sem0, sem1):
        bufs = (buf0, buf1)
        sems = (sem0, sem1)
        wid = lax.axis_index("s") * NC + lax.axis_index("c")
        pltpu.sync_copy(off_hbm, off_v)

        for kk in range(SEG_PER_W):
            seg = wid * SEG_PER_W + kk
            offv = off_v[pl.ds(seg, LANES)]
            a = offv[0]
            b = offv[1]
            n = b - a
            a8 = (a // 8) * 8
            nch = (b - a8 + CH - 1) // CH
            nquad = (nch + NBUF - 1) // NBUF

            def issue(c, buf, sem, a8=a8):
                row0 = jnp.minimum(a8 + c * CH, N - CH)
                pltpu.async_copy(fv_hbm.at[pl.ds(row0, CH)], buf, sem)

            def wait(buf, sem):
                pltpu.make_async_copy(
                    fv_hbm.at[pl.ds(0, CH)], buf, sem).wait()

            def process(buf, c, cy, a=a, b=b, a8=a8):
                row0 = a8 + c * CH
                w0 = jnp.minimum(row0, N - CH)
                lo = jnp.maximum(a, row0) - w0
                hi = jnp.maximum(jnp.minimum(b, row0 + CH) - w0, lo)
                cy = list(cy)
                for g in range(GROUPS):
                    sub = tuple(cy[3 * JPG * g: 3 * JPG * (g + 1)])

                    def row_body(r, sc, g=g, buf=buf):
                        out = []
                        for jj in range(JPG):
                            j = JPG * g + jj
                            v = buf[r, pl.ds(LANES * j, LANES)]
                            out += [
                                sc[3 * jj] + v,
                                jnp.minimum(sc[3 * jj + 1], v),
                                jnp.maximum(sc[3 * jj + 2], v),
                            ]
                        return tuple(out)

                    sub = plsc.parallel_loop(
                        lo, hi, 1, unroll=4, carry=sub)(row_body)
                    cy[3 * JPG * g: 3 * JPG * (g + 1)] = list(sub)
                return tuple(cy)

            carry = []
            for _ in range(NJ):
                carry += [
                    jnp.zeros((LANES,), jnp.float32),
                    jnp.full((LANES,), jnp.inf, jnp.float32),
                    jnp.full((LANES,), -jnp.inf, jnp.float32),
                ]
            carry = tuple(carry)

            for i in range(NBUF - 1):
                @pl.when(i < nch)
                def _(i=i):
                    issue(i, bufs[i], sems[i])

            def quad_body(q, cy):
                for i in range(NBUF):
                    c = NBUF * q + i

                    @pl.when(c < nch)
                    def _(c=c, i=i):
                        wait(bufs[i], sems[i])

                    @pl.when(c + (NBUF - 1) < nch)
                    def _(c=c, i=i):
                        issue(c + NBUF - 1, bufs[(i + NBUF - 1) % NBUF],
                              sems[(i + NBUF - 1) % NBUF])

                    cy = process(bufs[i], c, cy)
                return cy

            carry = lax.fori_loop(0, nquad, quad_body, carry)

            nf = n.astype(jnp.float32)
            inv = 1.0 / jnp.maximum(jnp.broadcast_to(nf, (LANES,)), 1.0)

            @pl.when(n > 0)
            def _(carry=carry, inv=inv, kk=kk):
                for j in range(NJ):
                    ds = pl.ds(kk * DV + LANES * j, LANES)
                    s_v[ds] = carry[3 * j] * inv
                    mn_v[ds] = carry[3 * j + 1]
                    mx_v[ds] = carry[3 * j + 2]

            @pl.when(n == 0)
            def _(kk=kk):
                z = jnp.zeros((LANES,), jnp.float32)
                for j in range(NJ):
                    ds = pl.ds(kk * DV + LANES * j, LANES)
                    s_v[ds] = z
                    mn_v[ds] = z
                    mx_v[ds] = z

        wbase = wid * SEG_PER_W * DV
        wlen = SEG_PER_W * DV
        pltpu.sync_copy(s_v, mean_hbm.at[pl.ds(wbase, wlen)])
        pltpu.sync_copy(mn_v, mn_hbm.at[pl.ds(wbase, wlen)])
        pltpu.sync_copy(mx_v, mx_hbm.at[pl.ds(wbase, wlen)])

    return k(fv, offsets)


def _tc_combine(mean, mn, mx, W1, W2, W3, bsum):
    def body(mean_ref, mn_ref, mx_ref, w1_ref, w2_ref, w3_ref, b_ref, o_ref):
        dn = (((1,), (1,)), ((), ()))
        acc = lax.dot_general(mean_ref[...].reshape(S, DV), w1_ref[...], dn,
                              precision=lax.Precision.HIGHEST,
                              preferred_element_type=jnp.float32)
        acc = acc + lax.dot_general(mn_ref[...].reshape(S, DV), w2_ref[...],
                                    dn, precision=lax.Precision.HIGHEST,
                                    preferred_element_type=jnp.float32)
        acc = acc + lax.dot_general(mx_ref[...].reshape(S, DV), w3_ref[...],
                                    dn, precision=lax.Precision.HIGHEST,
                                    preferred_element_type=jnp.float32)
        o_ref[...] = acc + b_ref[...]

    return pl.pallas_call(
        body,
        out_shape=jax.ShapeDtypeStruct((S, DG), jnp.float32),
    )(mean, mn, mx, W1, W2, W3, bsum)


def kernel(fv, segment_ids, W1, b1, W2, b2, W3, b3):
    ids = segment_ids.astype(jnp.int32)
    off = jnp.searchsorted(ids, jnp.arange(S + 1, dtype=jnp.int32),
                           side="left", method="compare_all").astype(jnp.int32)
    off = jnp.concatenate([off, jnp.zeros((OFF_PAD - (S + 1),), jnp.int32)])
    mean, mn, mx = _sc_segment_stats(fv, off)
    bsum = (b1 + b2 + b3).reshape(1, DG)
    return _tc_combine(mean, mn, mx, W1, W2, W3, bsum)
